# baseline (speedup 1.0000x reference)
import jax
import jax.numpy as jnp
from jax import lax
from jax.experimental import pallas as pl
from jax.experimental.pallas import tpu as pltpu
from jax.experimental.pallas import tpu_sc as plsc


def _bn_relu(x):
    mu = jnp.mean(x, axis=1, keepdims=True)
    var = jnp.mean((x - mu) * (x - mu), axis=1, keepdims=True)
    return jax.nn.relu((x - mu) / jnp.sqrt(var + 1e-5))


def _mlp(x, Ws, b_last):
    for W in Ws[:-1]:
        x = _bn_relu(jnp.dot(W, x, preferred_element_type=jnp.float32))
    return jnp.dot(Ws[-1], x, preferred_element_type=jnp.float32) + b_last


def _top3_S(dsq, size):
    iota = lax.broadcasted_iota(jnp.int32, dsq.shape, 0)
    key = lax.bitwise_or(
        lax.bitwise_and(lax.bitcast_convert_type(dsq, jnp.int32),
                        jnp.int32(-128)), iota)
    sels, ds = [], []
    for _ in range(3):
        kmin = jnp.min(key, axis=0, keepdims=True)
        sel = key == kmin
        sels.append(sel)
        ds.append(jnp.sqrt(lax.bitcast_convert_type(kmin, jnp.float32)))
        key = jnp.where(sel, jnp.int32(2147483647), key)
    sumd = ds[0] + ds[1] + ds[2]
    return (jnp.where(sels[0], 1.0 - ds[0] / sumd, 0.0)
            + jnp.where(sels[1], 1.0 - ds[1] / sumd, 0.0)
            + jnp.where(sels[2], 1.0 - ds[2] / sumd, 0.0))


def _prelude_body(nbf, igb, s32, gf, naf, iga, s16, na, nbt,
                  wba0, wba1, bba1,
                  wbp0, wbp1, wbp2, bbp2,
                  waa0, waa1, baa1,
                  wap0, wap1, wap2, bap2,
                  up_nb_ref, up_na_ref):
    Mb = nbf.shape[2]
    Ma = naf.shape[2]
    Cg = gf.shape[1]

    def catb(ref):
        return jnp.concatenate([ref[0], ref[1]], axis=1)

    nbf_c, igb_c = catb(nbf), catb(igb)
    att = _mlp(jnp.concatenate([nbf_c, igb_c], axis=0),
               [wba0[:], wba1[:]], bba1[:])
    att = jax.nn.softmax(att, axis=0)
    ws32 = jnp.concatenate(
        [jnp.dot(s32[0], att[:, :Mb], preferred_element_type=jnp.float32),
         jnp.dot(s32[1], att[:, Mb:], preferred_element_type=jnp.float32)],
        axis=1)
    gf_c = jnp.concatenate([jnp.broadcast_to(gf[0], (Cg, Mb)),
                            jnp.broadcast_to(gf[1], (Cg, Mb))], axis=1)
    up_nb = _mlp(jnp.concatenate([nbf_c, gf_c, ws32, igb_c], axis=0),
                 [wbp0[:], wbp1[:], wbp2[:]], bbp2[:])
    up_nb_ref[0] = up_nb[:, :Mb]
    up_nb_ref[1] = up_nb[:, Mb:]

    naf_c, iga_c = catb(naf), catb(iga)
    atta = _mlp(jnp.concatenate([naf_c, iga_c], axis=0),
                [waa0[:], waa1[:]], baa1[:])
    atta = jax.nn.softmax(atta, axis=0)
    ws16 = jnp.concatenate(
        [jnp.dot(s16[0], atta[:, :Ma], preferred_element_type=jnp.float32),
         jnp.dot(s16[1], atta[:, Ma:], preferred_element_type=jnp.float32)],
        axis=1)

    def interp_ab(b, ub):
        dsq = jnp.zeros((Mb, Ma), jnp.float32)
        for c in range(3):
            diff = nbt[b][:, c:c + 1] - na[b][c:c + 1, :]
            dsq = dsq + diff * diff
        S = _top3_S(dsq, Mb)
        return jnp.dot(ub, S, preferred_element_type=jnp.float32)

    iab_c = jnp.concatenate([interp_ab(0, up_nb[:, :Mb]),
                             interp_ab(1, up_nb[:, Mb:])], axis=1)
    up_na = _mlp(jnp.concatenate([naf_c, iab_c, ws16], axis=0),
                 [wap0[:], wap1[:], wap2[:]], bap2[:])
    up_na_ref[0] = up_na[:, :Ma]
    up_na_ref[1] = up_na[:, Ma:]


def _interp_pb_body(pc, nbt, up_nb, out_pb):
    Mb = nbt.shape[1]
    blk = pc.shape[2]
    pcb = pc[0]

    dsq = jnp.zeros((Mb, blk), jnp.float32)
    for c in range(3):
        diff = nbt[0][:, c:c + 1] - pcb[c:c + 1, :]
        dsq = dsq + diff * diff
    S = _top3_S(dsq, Mb)
    out_pb[0] = jnp.dot(up_nb[0], S, preferred_element_type=jnp.float32)


def _sc_sqrt(x):
    i = lax.bitcast_convert_type(x, jnp.int32)
    i = jnp.int32(0x5F3759DF) - lax.shift_right_logical(i, 1)
    y = lax.bitcast_convert_type(i, jnp.float32)
    for _ in range(3):
        y = y * (1.5 - 0.5 * x * y * y)
    return x * y


def _sc_pa_body(pc_hbm, naf_hbm, idxt_hbm, upf_hbm, out_hbm,
                tbl, nav, pcv, idxv, obuf):
    NC, NS = 2, 16
    wid = lax.axis_index("s") * NC + lax.axis_index("c")
    B = pc_hbm.shape[0]
    Ma = naf_hbm.shape[1] // 3
    Ca = upf_hbm.shape[1] // Ma
    npts = pc_hbm.shape[2] // (NC * NS)
    n0 = pl.multiple_of(wid * npts, 8)
    iota = lax.iota(jnp.int32, 16)

    for b in range(B):
        pltpu.sync_copy(upf_hbm.at[b], tbl)
        pltpu.sync_copy(naf_hbm.at[b], nav)
        pltpu.sync_copy(pc_hbm.at[b, :, pl.ds(n0, npts)], pcv)
        pltpu.sync_copy(idxt_hbm.at[b, :, pl.ds(n0, npts)], idxv)

        def wgroup(g, carry):
            s = pl.multiple_of(g * 16, 16)
            idxk = [idxv[k, pl.ds(s, 16)] for k in range(3)]
            pck = [pcv[c, pl.ds(s, 16)] for c in range(3)]
            ds = []
            for k in range(3):
                dsq = jnp.zeros((16,), jnp.float32)
                for c in range(3):
                    cc = plsc.load_gather(nav, [idxk[k] + (c * Ma)])
                    dd = pck[c] - cc
                    dsq = dsq + dd * dd
                ds.append(_sc_sqrt(dsq))
            sumd = ds[0] + ds[1] + ds[2]
            ws = [1.0 - d / sumd for d in ds]
            ixs = [idxk[k] * Ca for k in range(3)]

            @plsc.parallel_loop(0, Ca, unroll=4)
            def feat(f):
                acc = ws[0] * plsc.load_gather(tbl, [ixs[0] + f])
                acc = acc + ws[1] * plsc.load_gather(tbl, [ixs[1] + f])
                acc = acc + ws[2] * plsc.load_gather(tbl, [ixs[2] + f])
                plsc.store_scatter(obuf, [jnp.full((16,), f, jnp.int32),
                                          iota + s], acc)
            return carry

        jax.lax.fori_loop(0, npts // 16, wgroup, 0)
        pltpu.sync_copy(obuf, out_hbm.at[b, :, pl.ds(n0, npts)])


def kernel(pc, node_a, node_b, img_global_feature_BCMa, img_global_feature_BCMb,
           img_s32_feature_map_BCHw, img_s16_feature_map_BCHw, node_b_features,
           global_feature, node_a_features, node_a_min_k_idx, params):
    B, _, N = pc.shape
    Ma, Mb = node_a.shape[2], node_b.shape[2]
    C_b = params['node_b_pn'][-1][0].shape[0]
    C_a = params['node_a_pn'][-1][0].shape[0]

    nbt = jnp.transpose(node_b, (0, 2, 1))
    idxt = jnp.transpose(node_a_min_k_idx.astype(jnp.int32), (0, 2, 1))

    wbs = []
    for name in ('node_b_attention_pn', 'node_b_pn',
                 'node_a_attention_pn', 'node_a_pn'):
        layers = params[name]
        wbs += [W for (W, _) in layers]
        wbs.append(layers[-1][1].reshape(-1, 1))

    up_nb, up_na = pl.pallas_call(
        _prelude_body,
        out_shape=[jax.ShapeDtypeStruct((B, C_b, Mb), jnp.float32),
                   jax.ShapeDtypeStruct((B, C_a, Ma), jnp.float32)],
    )(node_b_features, img_global_feature_BCMb, img_s32_feature_map_BCHw,
      global_feature, node_a_features, img_global_feature_BCMa,
      img_s16_feature_map_BCHw, node_a, nbt, *wbs)

    NW = 32
    npts = N // NW
    interp_pa = pl.kernel(
        _sc_pa_body,
        out_type=jax.ShapeDtypeStruct((B, C_a, N), jnp.float32),
        mesh=plsc.VectorSubcoreMesh(core_axis_name="c", subcore_axis_name="s"),
        compiler_params=pltpu.CompilerParams(needs_layout_passes=False),
        scratch_types=[
            pltpu.VMEM((Ma * C_a,), jnp.float32),
            pltpu.VMEM((3 * Ma,), jnp.float32),
            pltpu.VMEM((3, npts), jnp.float32),
            pltpu.VMEM((3, npts), jnp.int32),
            pltpu.VMEM((C_a, npts), jnp.float32),
        ],
    )(pc, node_a.reshape(B, 3 * Ma), idxt,
      jnp.transpose(up_na, (0, 2, 1)).reshape(B, Ma * C_a))

    BLK = 4096
    interp_pb = pl.pallas_call(
        _interp_pb_body,
        grid=(B, N // BLK),
        in_specs=[
            pl.BlockSpec((1, 3, BLK), lambda b, i: (b, 0, i)),
            pl.BlockSpec((1, Mb, 3), lambda b, i: (b, 0, 0)),
            pl.BlockSpec((1, C_b, Mb), lambda b, i: (b, 0, 0)),
        ],
        out_specs=pl.BlockSpec((1, C_b, BLK), lambda b, i: (b, 0, i)),
        out_shape=jax.ShapeDtypeStruct((B, C_b, N), jnp.float32),
    )(pc, nbt, up_nb)

    return (interp_pa, interp_pb)


# speedup vs baseline: 2.9165x; 2.9165x over previous
import jax
import jax.numpy as jnp
from jax import lax
from jax.experimental import pallas as pl


def _bn_relu(x):
    mu = jnp.mean(x, axis=1, keepdims=True)
    var = jnp.mean((x - mu) * (x - mu), axis=1, keepdims=True)
    return jax.nn.relu((x - mu) / jnp.sqrt(var + 1e-5))


def _mlp(x, Ws, b_last):
    for W in Ws[:-1]:
        x = _bn_relu(jnp.dot(W, x.astype(jnp.bfloat16),
                             preferred_element_type=jnp.float32))
    return jnp.dot(Ws[-1], x.astype(jnp.bfloat16),
                   preferred_element_type=jnp.float32) + b_last


def _top3_S(dsq, size):
    iota = lax.broadcasted_iota(jnp.int32, dsq.shape, 0)
    key = lax.bitwise_or(
        lax.bitwise_and(lax.bitcast_convert_type(dsq, jnp.int32),
                        jnp.int32(-128)), iota)
    sels, ds = [], []
    for _ in range(3):
        kmin = jnp.min(key, axis=0, keepdims=True)
        sel = key == kmin
        sels.append(sel)
        ds.append(jnp.sqrt(lax.bitcast_convert_type(kmin, jnp.float32)))
        key = jnp.where(sel, jnp.int32(2147483647), key)
    sumd = ds[0] + ds[1] + ds[2]
    return (jnp.where(sels[0], 1.0 - ds[0] / sumd, 0.0)
            + jnp.where(sels[1], 1.0 - ds[1] / sumd, 0.0)
            + jnp.where(sels[2], 1.0 - ds[2] / sumd, 0.0))


def _prelude_body(nbf, igb, s32, gf, naf, iga, s16, na, nbt,
                  wba0, wba1, bba1,
                  wbp0, wbp1, wbp2, bbp2,
                  waa0, waa1, baa1,
                  wap0, wap1, wap2, bap2,
                  up_nb_ref, up_na_ref):
    Mb = nbf.shape[2]
    Ma = naf.shape[2]
    Cg = gf.shape[1]

    def catb(ref):
        return jnp.concatenate([ref[0], ref[1]], axis=1)

    nbf_c, igb_c = catb(nbf), catb(igb)
    att = _mlp(jnp.concatenate([nbf_c, igb_c], axis=0),
               [wba0[:], wba1[:]], bba1[:])
    att = jax.nn.softmax(att, axis=0)
    ws32 = jnp.concatenate(
        [jnp.dot(s32[0], att[:, :Mb], preferred_element_type=jnp.float32),
         jnp.dot(s32[1], att[:, Mb:], preferred_element_type=jnp.float32)],
        axis=1)
    gf_c = jnp.concatenate([jnp.broadcast_to(gf[0], (Cg, Mb)),
                            jnp.broadcast_to(gf[1], (Cg, Mb))], axis=1)
    up_nb = _mlp(jnp.concatenate([nbf_c, gf_c, ws32, igb_c], axis=0),
                 [wbp0[:], wbp1[:], wbp2[:]], bbp2[:])
    up_nb_ref[0] = up_nb[:, :Mb]
    up_nb_ref[1] = up_nb[:, Mb:]

    naf_c, iga_c = catb(naf), catb(iga)
    atta = _mlp(jnp.concatenate([naf_c, iga_c], axis=0),
                [waa0[:], waa1[:]], baa1[:])
    atta = jax.nn.softmax(atta, axis=0)
    ws16 = jnp.concatenate(
        [jnp.dot(s16[0], atta[:, :Ma], preferred_element_type=jnp.float32),
         jnp.dot(s16[1], atta[:, Ma:], preferred_element_type=jnp.float32)],
        axis=1)

    def interp_ab(b, ub):
        dsq = jnp.zeros((Mb, Ma), jnp.float32)
        for c in range(3):
            diff = nbt[b][:, c:c + 1] - na[b][c:c + 1, :]
            dsq = dsq + diff * diff
        S = _top3_S(dsq, Mb)
        return jnp.dot(ub, S, preferred_element_type=jnp.float32)

    iab_c = jnp.concatenate([interp_ab(0, up_nb[:, :Mb]),
                             interp_ab(1, up_nb[:, Mb:])], axis=1)
    up_na = _mlp(jnp.concatenate([naf_c, iab_c, ws16], axis=0),
                 [wap0[:], wap1[:], wap2[:]], bap2[:])
    up_na_ref[0] = up_na[:, :Ma]
    up_na_ref[1] = up_na[:, Ma:]


def _interp_body(pc, nbt, na, idxt, up_nb, up_na, out_pb, out_pa):
    Mb = nbt.shape[1]
    Ma = na.shape[2]
    blk = pc.shape[2]
    pcb = pc[0]

    dsq = jnp.zeros((Mb, blk), jnp.float32)
    for c in range(3):
        diff = nbt[0][:, c:c + 1] - pcb[c:c + 1, :]
        dsq = dsq + diff * diff
    S = _top3_S(dsq, Mb)
    out_pb[0] = jnp.dot(up_nb[0], S, preferred_element_type=jnp.float32)

    idx = idxt[0]
    naa = na[0]
    iota = lax.broadcasted_iota(jnp.int32, (Ma, blk), 0)
    ms, ds = [], []
    for k in range(3):
        m = iota == idx[k:k + 1, :]
        oh = jnp.where(m, 1.0, 0.0)
        tb = jnp.dot(naa, oh, preferred_element_type=jnp.float32)
        dd = pcb - tb
        ds.append(jnp.sqrt(jnp.sum(dd * dd, axis=0, keepdims=True)))
        ms.append(m)
    sumd = ds[0] + ds[1] + ds[2]
    Sa = (jnp.where(ms[0], 1.0 - ds[0] / sumd, 0.0)
          + jnp.where(ms[1], 1.0 - ds[1] / sumd, 0.0)
          + jnp.where(ms[2], 1.0 - ds[2] / sumd, 0.0))
    out_pa[0] = jnp.dot(up_na[0], Sa, preferred_element_type=jnp.float32)


def kernel(pc, node_a, node_b, img_global_feature_BCMa, img_global_feature_BCMb,
           img_s32_feature_map_BCHw, img_s16_feature_map_BCHw, node_b_features,
           global_feature, node_a_features, node_a_min_k_idx, params):
    B, _, N = pc.shape
    Ma, Mb = node_a.shape[2], node_b.shape[2]
    C_b = params['node_b_pn'][-1][0].shape[0]
    C_a = params['node_a_pn'][-1][0].shape[0]

    nbt = jnp.transpose(node_b, (0, 2, 1))
    idxt = jnp.transpose(node_a_min_k_idx.astype(jnp.int32), (0, 2, 1))

    wbs = []
    for name in ('node_b_attention_pn', 'node_b_pn',
                 'node_a_attention_pn', 'node_a_pn'):
        layers = params[name]
        wbs += [W.astype(jnp.bfloat16) for (W, _) in layers]
        wbs.append(layers[-1][1].reshape(-1, 1))

    up_nb, up_na = pl.pallas_call(
        _prelude_body,
        out_shape=[jax.ShapeDtypeStruct((B, C_b, Mb), jnp.float32),
                   jax.ShapeDtypeStruct((B, C_a, Ma), jnp.float32)],
    )(node_b_features, img_global_feature_BCMb, img_s32_feature_map_BCHw,
      global_feature, node_a_features, img_global_feature_BCMa,
      img_s16_feature_map_BCHw, node_a, nbt, *wbs)

    BLK = 4096
    interp_pb, interp_pa = pl.pallas_call(
        _interp_body,
        grid=(B, N // BLK),
        in_specs=[
            pl.BlockSpec((1, 3, BLK), lambda b, i: (b, 0, i)),
            pl.BlockSpec((1, Mb, 3), lambda b, i: (b, 0, 0)),
            pl.BlockSpec((1, 3, Ma), lambda b, i: (b, 0, 0)),
            pl.BlockSpec((1, 3, BLK), lambda b, i: (b, 0, i)),
            pl.BlockSpec((1, C_b, Mb), lambda b, i: (b, 0, 0)),
            pl.BlockSpec((1, C_a, Ma), lambda b, i: (b, 0, 0)),
        ],
        out_specs=[
            pl.BlockSpec((1, C_b, BLK), lambda b, i: (b, 0, i)),
            pl.BlockSpec((1, C_a, BLK), lambda b, i: (b, 0, i)),
        ],
        out_shape=[jax.ShapeDtypeStruct((B, C_b, N), jnp.float32),
                   jax.ShapeDtypeStruct((B, C_a, N), jnp.float32)],
    )(pc, nbt, node_a, idxt, up_nb, up_na)

    return (interp_pa, interp_pb)


# speedup vs baseline: 3.4908x; 1.1969x over previous
import jax
import jax.numpy as jnp
from jax import lax
from jax.experimental import pallas as pl


def _bn_relu(x):
    mu = jnp.mean(x, axis=1, keepdims=True)
    var = jnp.mean((x - mu) * (x - mu), axis=1, keepdims=True)
    return jax.nn.relu((x - mu) / jnp.sqrt(var + 1e-5))


def _mlp(x, Ws, b_last):
    for W in Ws[:-1]:
        x = _bn_relu(jnp.dot(W, x, preferred_element_type=jnp.float32))
    y = jnp.dot(Ws[-1], x, preferred_element_type=jnp.float32)
    return y + b_last.reshape(-1, 1)


def _top3_S(dsq, size):
    iota = lax.broadcasted_iota(jnp.int32, dsq.shape, 0)
    key = lax.bitwise_or(
        lax.bitwise_and(lax.bitcast_convert_type(dsq, jnp.int32),
                        jnp.int32(-128)), iota)
    sels, ds = [], []
    for _ in range(3):
        kmin = jnp.min(key, axis=0, keepdims=True)
        sel = key == kmin
        sels.append(sel)
        ds.append(jnp.sqrt(lax.bitcast_convert_type(kmin, jnp.float32)))
        key = jnp.where(sel, jnp.int32(2147483647), key)
    sumd = ds[0] + ds[1] + ds[2]
    return (jnp.where(sels[0], 1.0 - ds[0] / sumd, 0.0)
            + jnp.where(sels[1], 1.0 - ds[1] / sumd, 0.0)
            + jnp.where(sels[2], 1.0 - ds[2] / sumd, 0.0))


def _prelude_body(nbf, igb, s32, gf, naf, iga, s16, na, nbt,
                  wba0, wba1, bba1,
                  wbp0, wbp1, wbp2, bbp2,
                  waa0, waa1, baa1,
                  wap0, wap1, wap2, bap2,
                  up_nb_ref, up_na_ref):
    Mb = nbf.shape[2]
    Ma = naf.shape[2]
    Cg = gf.shape[1]

    def catb(ref):
        return jnp.concatenate([ref[0], ref[1]], axis=1)

    nbf_c, igb_c = catb(nbf), catb(igb)
    att = _mlp(jnp.concatenate([nbf_c, igb_c], axis=0),
               [wba0[:], wba1[:]], bba1[:])
    att = jax.nn.softmax(att, axis=0)
    ws32 = jnp.concatenate(
        [jnp.dot(s32[0], att[:, :Mb], preferred_element_type=jnp.float32),
         jnp.dot(s32[1], att[:, Mb:], preferred_element_type=jnp.float32)],
        axis=1)
    gf_c = jnp.concatenate([jnp.broadcast_to(gf[0], (Cg, Mb)),
                            jnp.broadcast_to(gf[1], (Cg, Mb))], axis=1)
    up_nb = _mlp(jnp.concatenate([nbf_c, gf_c, ws32, igb_c], axis=0),
                 [wbp0[:], wbp1[:], wbp2[:]], bbp2[:])
    up_nb_ref[0] = up_nb[:, :Mb]
    up_nb_ref[1] = up_nb[:, Mb:]

    naf_c, iga_c = catb(naf), catb(iga)
    atta = _mlp(jnp.concatenate([naf_c, iga_c], axis=0),
                [waa0[:], waa1[:]], baa1[:])
    atta = jax.nn.softmax(atta, axis=0)
    ws16 = jnp.concatenate(
        [jnp.dot(s16[0], atta[:, :Ma], preferred_element_type=jnp.float32),
         jnp.dot(s16[1], atta[:, Ma:], preferred_element_type=jnp.float32)],
        axis=1)

    def interp_ab(b, ub):
        dsq = jnp.zeros((Mb, Ma), jnp.float32)
        for c in range(3):
            diff = nbt[b][:, c:c + 1] - na[b][c:c + 1, :]
            dsq = dsq + diff * diff
        S = _top3_S(dsq, Mb)
        return jnp.dot(ub, S, preferred_element_type=jnp.float32)

    iab_c = jnp.concatenate([interp_ab(0, up_nb[:, :Mb]),
                             interp_ab(1, up_nb[:, Mb:])], axis=1)
    up_na = _mlp(jnp.concatenate([naf_c, iab_c, ws16], axis=0),
                 [wap0[:], wap1[:], wap2[:]], bap2[:])
    up_na_ref[0] = up_na[:, :Ma]
    up_na_ref[1] = up_na[:, Ma:]


def _interp_body(pc, nbt, na, idxt, up_nb, up_na, out_pb, out_pa):
    Mb = nbt.shape[1]
    Ma = na.shape[2]
    blk = pc.shape[2]
    pcb = pc[0]

    dsq = jnp.zeros((Mb, blk), jnp.float32)
    for c in range(3):
        diff = nbt[0][:, c:c + 1] - pcb[c:c + 1, :]
        dsq = dsq + diff * diff
    S = _top3_S(dsq, Mb)
    out_pb[0] = jnp.dot(up_nb[0], S, preferred_element_type=jnp.float32)

    idx = idxt[0]
    naa = na[0]
    iota = lax.broadcasted_iota(jnp.int32, (Ma, blk), 0)
    ms, ds = [], []
    for k in range(3):
        m = iota == idx[k:k + 1, :]
        oh = jnp.where(m, 1.0, 0.0)
        tb = jnp.dot(naa, oh, preferred_element_type=jnp.float32)
        dd = pcb - tb
        ds.append(jnp.sqrt(jnp.sum(dd * dd, axis=0, keepdims=True)))
        ms.append(m)
    sumd = ds[0] + ds[1] + ds[2]
    Sa = (jnp.where(ms[0], 1.0 - ds[0] / sumd, 0.0)
          + jnp.where(ms[1], 1.0 - ds[1] / sumd, 0.0)
          + jnp.where(ms[2], 1.0 - ds[2] / sumd, 0.0))
    out_pa[0] = jnp.dot(up_na[0], Sa, preferred_element_type=jnp.float32)


def kernel(pc, node_a, node_b, img_global_feature_BCMa, img_global_feature_BCMb,
           img_s32_feature_map_BCHw, img_s16_feature_map_BCHw, node_b_features,
           global_feature, node_a_features, node_a_min_k_idx, params):
    B, _, N = pc.shape
    Ma, Mb = node_a.shape[2], node_b.shape[2]
    C_b = params['node_b_pn'][-1][0].shape[0]
    C_a = params['node_a_pn'][-1][0].shape[0]

    nbt = jnp.transpose(node_b, (0, 2, 1))
    idxt = jnp.transpose(node_a_min_k_idx.astype(jnp.int32), (0, 2, 1))

    wbs = []
    for name in ('node_b_attention_pn', 'node_b_pn',
                 'node_a_attention_pn', 'node_a_pn'):
        layers = params[name]
        wbs += [W for (W, _) in layers]
        wbs.append(layers[-1][1])

    up_nb, up_na = pl.pallas_call(
        _prelude_body,
        out_shape=[jax.ShapeDtypeStruct((B, C_b, Mb), jnp.float32),
                   jax.ShapeDtypeStruct((B, C_a, Ma), jnp.float32)],
    )(node_b_features, img_global_feature_BCMb, img_s32_feature_map_BCHw,
      global_feature, node_a_features, img_global_feature_BCMa,
      img_s16_feature_map_BCHw, node_a, nbt, *wbs)

    BLK = 4096
    interp_pb, interp_pa = pl.pallas_call(
        _interp_body,
        grid=(B, N // BLK),
        in_specs=[
            pl.BlockSpec((1, 3, BLK), lambda b, i: (b, 0, i)),
            pl.BlockSpec((1, Mb, 3), lambda b, i: (b, 0, 0)),
            pl.BlockSpec((1, 3, Ma), lambda b, i: (b, 0, 0)),
            pl.BlockSpec((1, 3, BLK), lambda b, i: (b, 0, i)),
            pl.BlockSpec((1, C_b, Mb), lambda b, i: (b, 0, 0)),
            pl.BlockSpec((1, C_a, Ma), lambda b, i: (b, 0, 0)),
        ],
        out_specs=[
            pl.BlockSpec((1, C_b, BLK), lambda b, i: (b, 0, i)),
            pl.BlockSpec((1, C_a, BLK), lambda b, i: (b, 0, i)),
        ],
        out_shape=[jax.ShapeDtypeStruct((B, C_b, N), jnp.float32),
                   jax.ShapeDtypeStruct((B, C_a, N), jnp.float32)],
    )(pc, nbt, node_a, idxt, up_nb, up_na)

    return (interp_pa, interp_pb)
